# baseline (speedup 1.0000x reference)
import jax
import jax.numpy as jnp
from jax import lax
from jax.experimental import pallas as pl
from jax.experimental.pallas import tpu as pltpu
from jax.experimental.pallas import tpu_sc as plsc

IOU_THRESHOLD = 0.5
POS_WEIGHT = 0.5

B, T, S, A = 64, 100, 3, 3
HS = (64, 32, 16)
N_TOTAL = B * A * sum(h * h for h in HS)
N_UPD = B * T * S * A
ROW = 120
N_ROWS = N_UPD // ROW
ROWS_PER_SCALE = B * T * A // ROW
NC, NS = 2, 16
ROWS_PER_WORKER = ROWS_PER_SCALE // (NC * NS)


def _softplus(x):
    return jnp.maximum(x, 0.0) + jnp.log1p(jnp.exp(-jnp.abs(x)))


def _prep_body(tgt_ref, anch_ref, idx_ref, val_ref):
    tgt = tgt_ref[...]
    tx, ty, tw, th = tgt[:, 0, :], tgt[:, 1, :], tgt[:, 2, :], tgt[:, 3, :]
    mask = (tx != -1.0) & (ty != -1.0) & (tw != -1.0) & (th != -1.0)
    biota = lax.broadcasted_iota(jnp.int32, (B, T), 0)

    zx, zy, zw, zh, ta, cy, cx, valid = [], [], [], [], [], [], [], []
    for s in range(S):
        h = float(HS[s])
        sx = tx * h
        sy = ty * h
        zx.append(sx - (jnp.floor(sx) + 0.5))
        zy.append(sy - (jnp.floor(sy) + 0.5))
        zw.append(tw * h)
        zh.append(th * h)
        ta.append((zw[s] - zx[s]) * (zh[s] - zy[s]))
        _cy = sy.astype(jnp.int32)
        _cx = sx.astype(jnp.int32)
        hi = HS[s]
        cy.append(_cy)
        cx.append(_cx)
        valid.append(mask & (_cy >= 0) & (_cy < hi) & (_cx >= 0) & (_cx < hi))

    dy = cy[0][:, :, None] ^ cy[0][:, None, :]
    dx = cx[0][:, :, None] ^ cx[0][:, None, :]
    later = (lax.broadcasted_iota(jnp.int32, (B, T, T), 2)
             > lax.broadcasted_iota(jnp.int32, (B, T, T), 1))
    live_base = []
    for s in range(S):
        same = ((dy >> s) | (dx >> s)) == 0
        killer = jnp.broadcast_to(valid[s][:, None, :], (B, T, T))
        killed = jnp.any(same & later & killer, axis=2)
        live_base.append(valid[s] & jnp.logical_not(killed))

    ious = []
    for s in range(S):
        for a in range(A):
            aw = anch_ref[0, (s * A + a) * 2]
            ah = anch_ref[0, (s * A + a) * 2 + 1]
            lx = jnp.maximum(-aw / 2, zx[s] - zw[s] / 2)
            ux = jnp.minimum(aw / 2, zx[s] + zw[s] / 2)
            ly = jnp.maximum(-ah / 2, zy[s] - zh[s] / 2)
            uy = jnp.minimum(ah / 2, zy[s] + zh[s] / 2)
            m = ((lx < ux).astype(jnp.float32) * (ly < uy).astype(jnp.float32))
            inter = (ux - lx) * (uy - ly) * m
            ious.append(inter / ((aw * ah + ta[s]) - inter))
    mx = ious[0]
    for z in ious[1:]:
        mx = jnp.maximum(mx, z)

    for s in range(S):
        hi = HS[s]
        for a in range(A):
            v = ious[s * A + a]
            v = jnp.where(v == mx, 1.0, v)
            v = jnp.where(v < IOU_THRESHOLD, 0.0, v)
            v = jnp.where((v != 0.0) & (v != 1.0), -1.0, v)
            if s == 0:
                addr = (((biota * A + a) * hi + cy[s]) * 128) + cx[s]
            else:
                addr = (((a * hi + cy[s]) * hi + cx[s]) * 128) + biota
            idx_ref[s * A + a] = jnp.where(valid[s], addr, 0)
            val_ref[s * A + a] = jnp.where(live_base[s], v, 0.0)


_CHUNK = 16
_NCHUNKS = (S * A * B) // _CHUNK


def _gather_body(d0_hbm, d1_hbm, d2_hbm, idx_hbm, xsel_hbm, idx_v, out_v, sem):
    c = lax.axis_index("c")
    sub = lax.axis_index("s")
    wid = sub * NC + c
    tables = (d0_hbm, d1_hbm, d2_hbm)
    for ck in range(_NCHUNKS):
        scale = ck // (_NCHUNKS // S)
        r0 = ck * _CHUNK

        @pl.when((wid == ck) | (wid == ck - (NC * NS)))
        def _(scale=scale, r0=r0):
            tbl = tables[scale]
            pltpu.sync_copy(idx_hbm.at[pl.ds(r0, _CHUNK)], idx_v)
            cps = [pltpu.async_copy(tbl.at[idx_v.at[r]], out_v.at[r], sem)
                   for r in range(_CHUNK)]
            for cp in cps:
                cp.wait()
            pltpu.sync_copy(out_v, xsel_hbm.at[pl.ds(r0, _CHUNK)])


def _gather(d0f, d1f, d2f, idx2):
    mesh = plsc.VectorSubcoreMesh(core_axis_name="c", subcore_axis_name="s")
    return pl.kernel(
        _gather_body,
        out_type=jax.ShapeDtypeStruct((S * A * B, T), jnp.float32),
        mesh=mesh,
        scratch_types=[
            pltpu.VMEM((_CHUNK, T), jnp.int32),
            pltpu.VMEM((_CHUNK, T), jnp.float32),
            pltpu.SemaphoreType.DMA,
        ],
    )(d0f, d1f, d2f, idx2)


_GRID = 16
_BB = B // _GRID


def _tables_kernel(d0_ref, d1_ref, d2_ref, tgt_ref, anch_ref, lg0_ref,
                   lg1_ref, lg2_ref, idx_ref, val_ref):
    i = pl.program_id(0)
    x = d0_ref[...]
    lg0_ref[:, :, :, :, 0:x.shape[4]] = x

    @pl.when(i == 0)
    def _():
        for ref, lg in ((d1_ref, lg1_ref), (d2_ref, lg2_ref)):
            y = ref[...]
            lg[:, :, :, :, 0:y.shape[4]] = y
        _prep_body(tgt_ref, anch_ref, idx_ref, val_ref)


def _tables(d0t, d1t, d2t, tgt, anch):
    h0, h1, h2 = HS
    return pl.pallas_call(
        _tables_kernel,
        grid=(_GRID,),
        out_shape=(
            jax.ShapeDtypeStruct((B, A, 1, h0, 128), jnp.float32),
            jax.ShapeDtypeStruct((A, h1, 1, h1, 128), jnp.float32),
            jax.ShapeDtypeStruct((A, h2, 1, h2, 128), jnp.float32),
            jax.ShapeDtypeStruct((S * A, B, T), jnp.int32),
            jax.ShapeDtypeStruct((S * A, B, T), jnp.float32),
        ),
        in_specs=[
            pl.BlockSpec((_BB, A, 1, h0, h0), lambda i: (i, 0, 4, 0, 0)),
            pl.BlockSpec((A, h1, 1, h1, B), lambda i: (0, 0, 4, 0, 0)),
            pl.BlockSpec((A, h2, 1, h2, B), lambda i: (0, 0, 4, 0, 0)),
            pl.BlockSpec((B, 4, T), lambda i: (0, 0, 0)),
            pl.BlockSpec(memory_space=pltpu.SMEM),
        ],
        out_specs=(
            pl.BlockSpec((_BB, A, 1, h0, 128), lambda i: (i, 0, 0, 0, 0)),
            pl.BlockSpec((A, h1, 1, h1, 128), lambda i: (0, 0, 0, 0, 0)),
            pl.BlockSpec((A, h2, 1, h2, 128), lambda i: (0, 0, 0, 0, 0)),
            pl.BlockSpec((S * A, B, T), lambda i: (0, 0, 0)),
            pl.BlockSpec((S * A, B, T), lambda i: (0, 0, 0)),
        ),
    )(d0t, d1t, d2t, tgt, anch)


def _sums_kernel(lg0_ref, lg1_ref, lg2_ref, acc_ref):
    i = pl.program_id(0)
    x = lg0_ref[...]
    lane = lax.broadcasted_iota(jnp.int32, x.shape, 4)
    acc = jnp.sum(jnp.where(lane < B, _softplus(x), 0.0),
                  keepdims=True).reshape(1, 1)

    @pl.when(i == 0)
    def _():
        extra = jnp.zeros((1, 1), jnp.float32)
        for ref in (lg1_ref, lg2_ref):
            y = ref[...]
            ln = lax.broadcasted_iota(jnp.int32, y.shape, 4)
            extra += jnp.sum(jnp.where(ln < B, _softplus(y), 0.0),
                             keepdims=True).reshape(1, 1)
        acc_ref[...] = extra

    acc_ref[...] += acc


def _sums(lg0, lg1, lg2):
    h0, h1, h2 = HS
    return pl.pallas_call(
        _sums_kernel,
        grid=(_GRID,),
        out_shape=jax.ShapeDtypeStruct((1, 1), jnp.float32),
        in_specs=[
            pl.BlockSpec((_BB, A, 1, h0, 128), lambda i: (i, 0, 0, 0, 0)),
            pl.BlockSpec((A, h1, 1, h1, 128), lambda i: (0, 0, 0, 0, 0)),
            pl.BlockSpec((A, h2, 1, h2, 128), lambda i: (0, 0, 0, 0, 0)),
        ],
        out_specs=pl.BlockSpec((1, 1), lambda i: (0, 0)),
    )(lg0, lg1, lg2)


def _final_kernel(acc_ref, xs_ref, vv_ref, out_ref):
    x = xs_ref[...]
    corr = vv_ref[...] * (POS_WEIGHT * _softplus(-x) - _softplus(x))
    total = acc_ref[...] + jnp.sum(corr, keepdims=True).reshape(1, 1)
    out_ref[...] = total / jnp.float32(N_TOTAL)


def _final(acc, xsel, val):
    return pl.pallas_call(
        _final_kernel,
        out_shape=jax.ShapeDtypeStruct((1, 1), jnp.float32),
        in_specs=[
            pl.BlockSpec((1, 1), lambda: (0, 0)),
            pl.BlockSpec((S * A * B, T), lambda: (0, 0)),
            pl.BlockSpec((S * A * B, T), lambda: (0, 0)),
        ],
        out_specs=pl.BlockSpec((1, 1), lambda: (0, 0)),
    )(acc, xsel, val)


def kernel(det0, det1, det2, scaled_anchors, targets):
    anch = scaled_anchors.reshape(1, S * A * 2)
    lg0, lg1, lg2, idx, val = _tables(det0.transpose(0, 1, 4, 2, 3),
                                      jnp.transpose(det1, (1, 2, 4, 3, 0)),
                                      jnp.transpose(det2, (1, 2, 4, 3, 0)),
                                      jnp.transpose(targets, (0, 2, 1)), anch)
    xsel = _gather(lg0.reshape(-1), lg1.reshape(-1), lg2.reshape(-1),
                   idx.reshape(S * A * B, T))
    acc = _sums(lg0, lg1, lg2)
    loss = _final(acc, xsel, val.reshape(S * A * B, T))
    return loss[0, 0]


# speedup vs baseline: 1.2357x; 1.2357x over previous
import jax
import jax.numpy as jnp
from jax import lax
from jax.experimental import pallas as pl
from jax.experimental.pallas import tpu as pltpu
from jax.experimental.pallas import tpu_sc as plsc

IOU_THRESHOLD = 0.5
POS_WEIGHT = 0.5

B, T, S, A = 64, 100, 3, 3
HS = (64, 32, 16)
N_TOTAL = B * A * sum(h * h for h in HS)
N_UPD = B * T * S * A
ROW = 120
N_ROWS = N_UPD // ROW
ROWS_PER_SCALE = B * T * A // ROW
NC, NS = 2, 16
ROWS_PER_WORKER = ROWS_PER_SCALE // (NC * NS)


def _softplus(x):
    return jnp.maximum(x, 0.0) + jnp.log1p(jnp.exp(-jnp.abs(x)))


def _prep_body(tgt_ref, anch_ref, idx_ref, val_ref):
    tgt = tgt_ref[...]
    tx, ty, tw, th = tgt[:, 0, :], tgt[:, 1, :], tgt[:, 2, :], tgt[:, 3, :]
    mask = (tx != -1.0) & (ty != -1.0) & (tw != -1.0) & (th != -1.0)
    biota = lax.broadcasted_iota(jnp.int32, (B, T), 0)

    zx, zy, zw, zh, ta, cy, cx, valid = [], [], [], [], [], [], [], []
    for s in range(S):
        h = float(HS[s])
        sx = tx * h
        sy = ty * h
        zx.append(sx - (jnp.floor(sx) + 0.5))
        zy.append(sy - (jnp.floor(sy) + 0.5))
        zw.append(tw * h)
        zh.append(th * h)
        ta.append((zw[s] - zx[s]) * (zh[s] - zy[s]))
        _cy = sy.astype(jnp.int32)
        _cx = sx.astype(jnp.int32)
        hi = HS[s]
        cy.append(_cy)
        cx.append(_cx)
        valid.append(mask & (_cy >= 0) & (_cy < hi) & (_cx >= 0) & (_cx < hi))

    dy = cy[0][:, :, None] ^ cy[0][:, None, :]
    dx = cx[0][:, :, None] ^ cx[0][:, None, :]
    later = (lax.broadcasted_iota(jnp.int32, (B, T, T), 2)
             > lax.broadcasted_iota(jnp.int32, (B, T, T), 1))
    live_base = []
    for s in range(S):
        same = ((dy >> s) | (dx >> s)) == 0
        killer = jnp.broadcast_to(valid[s][:, None, :], (B, T, T))
        killed = jnp.any(same & later & killer, axis=2)
        live_base.append(valid[s] & jnp.logical_not(killed))

    ious = []
    for s in range(S):
        for a in range(A):
            aw = anch_ref[0, (s * A + a) * 2]
            ah = anch_ref[0, (s * A + a) * 2 + 1]
            lx = jnp.maximum(-aw / 2, zx[s] - zw[s] / 2)
            ux = jnp.minimum(aw / 2, zx[s] + zw[s] / 2)
            ly = jnp.maximum(-ah / 2, zy[s] - zh[s] / 2)
            uy = jnp.minimum(ah / 2, zy[s] + zh[s] / 2)
            m = ((lx < ux).astype(jnp.float32) * (ly < uy).astype(jnp.float32))
            inter = (ux - lx) * (uy - ly) * m
            ious.append(inter / ((aw * ah + ta[s]) - inter))
    mx = ious[0]
    for z in ious[1:]:
        mx = jnp.maximum(mx, z)

    for s in range(S):
        hi = HS[s]
        for a in range(A):
            v = ious[s * A + a]
            v = jnp.where(v == mx, 1.0, v)
            v = jnp.where(v < IOU_THRESHOLD, 0.0, v)
            v = jnp.where((v != 0.0) & (v != 1.0), -1.0, v)
            if s == 0:
                addr = (((biota * A + a) * hi + cy[s]) * 128) + cx[s]
            else:
                addr = (((a * hi + cy[s]) * hi + cx[s]) * 128) + biota
            idx_ref[s * A + a] = jnp.where(valid[s], addr, 0)
            val_ref[s * A + a] = jnp.where(live_base[s], v, 0.0)


_CHUNK = 16
_NCHUNKS = (S * A * B) // _CHUNK


def _gather_body(d0_hbm, d1_hbm, d2_hbm, idx_hbm, xsel_hbm, idx_v, out_v, sem):
    c = lax.axis_index("c")
    sub = lax.axis_index("s")
    wid = sub * NC + c
    tables = (d0_hbm, d1_hbm, d2_hbm)
    for ck in range(_NCHUNKS):
        scale = ck // (_NCHUNKS // S)
        r0 = ck * _CHUNK

        @pl.when((wid == ck) | (wid == ck - (NC * NS)))
        def _(scale=scale, r0=r0):
            tbl = tables[scale]
            pltpu.sync_copy(idx_hbm.at[pl.ds(r0, _CHUNK)], idx_v)
            cps = [pltpu.async_copy(tbl.at[idx_v.at[r]], out_v.at[r], sem)
                   for r in range(_CHUNK)]
            for cp in cps:
                cp.wait()
            pltpu.sync_copy(out_v, xsel_hbm.at[pl.ds(r0, _CHUNK)])


def _gather(d0f, d1f, d2f, idx2):
    mesh = plsc.VectorSubcoreMesh(core_axis_name="c", subcore_axis_name="s")
    return pl.kernel(
        _gather_body,
        out_type=jax.ShapeDtypeStruct((S * A * B, T), jnp.float32),
        mesh=mesh,
        scratch_types=[
            pltpu.VMEM((_CHUNK, T), jnp.int32),
            pltpu.VMEM((_CHUNK, T), jnp.float32),
            pltpu.SemaphoreType.DMA,
        ],
    )(d0f, d1f, d2f, idx2)


_GRID = 4
_BB = B // _GRID


def _tables_kernel(d0_ref, d1_ref, d2_ref, tgt_ref, anch_ref, lg0_ref,
                   lg1_ref, lg2_ref, idx_ref, val_ref):
    i = pl.program_id(0)
    x = d0_ref[...]
    lg0_ref[:, :, :, :, 0:x.shape[4]] = x

    @pl.when(i == 0)
    def _():
        for ref, lg in ((d1_ref, lg1_ref), (d2_ref, lg2_ref)):
            y = ref[...]
            lg[:, :, :, :, 0:y.shape[4]] = y
        _prep_body(tgt_ref, anch_ref, idx_ref, val_ref)


def _tables(d0t, d1t, d2t, tgt, anch):
    h0, h1, h2 = HS
    return pl.pallas_call(
        _tables_kernel,
        grid=(_GRID,),
        out_shape=(
            jax.ShapeDtypeStruct((B, A, 1, h0, 128), jnp.float32),
            jax.ShapeDtypeStruct((A, h1, 1, h1, 128), jnp.float32),
            jax.ShapeDtypeStruct((A, h2, 1, h2, 128), jnp.float32),
            jax.ShapeDtypeStruct((S * A, B, T), jnp.int32),
            jax.ShapeDtypeStruct((S * A, B, T), jnp.float32),
        ),
        in_specs=[
            pl.BlockSpec((_BB, A, 1, h0, h0), lambda i: (i, 0, 4, 0, 0)),
            pl.BlockSpec((A, h1, 1, h1, B), lambda i: (0, 0, 4, 0, 0)),
            pl.BlockSpec((A, h2, 1, h2, B), lambda i: (0, 0, 4, 0, 0)),
            pl.BlockSpec((B, 4, T), lambda i: (0, 0, 0)),
            pl.BlockSpec(memory_space=pltpu.SMEM),
        ],
        out_specs=(
            pl.BlockSpec((_BB, A, 1, h0, 128), lambda i: (i, 0, 0, 0, 0)),
            pl.BlockSpec((A, h1, 1, h1, 128), lambda i: (0, 0, 0, 0, 0)),
            pl.BlockSpec((A, h2, 1, h2, 128), lambda i: (0, 0, 0, 0, 0)),
            pl.BlockSpec((S * A, B, T), lambda i: (0, 0, 0)),
            pl.BlockSpec((S * A, B, T), lambda i: (0, 0, 0)),
        ),
    )(d0t, d1t, d2t, tgt, anch)


def _sums_kernel(lg0_ref, lg1_ref, lg2_ref, acc_ref):
    i = pl.program_id(0)
    x = lg0_ref[...]
    lane = lax.broadcasted_iota(jnp.int32, x.shape, 4)
    acc = jnp.sum(jnp.where(lane < B, _softplus(x), 0.0),
                  keepdims=True).reshape(1, 1)

    @pl.when(i == 0)
    def _():
        extra = jnp.zeros((1, 1), jnp.float32)
        for ref in (lg1_ref, lg2_ref):
            y = ref[...]
            ln = lax.broadcasted_iota(jnp.int32, y.shape, 4)
            extra += jnp.sum(jnp.where(ln < B, _softplus(y), 0.0),
                             keepdims=True).reshape(1, 1)
        acc_ref[...] = extra

    acc_ref[...] += acc


def _sums(lg0, lg1, lg2):
    h0, h1, h2 = HS
    return pl.pallas_call(
        _sums_kernel,
        grid=(_GRID,),
        out_shape=jax.ShapeDtypeStruct((1, 1), jnp.float32),
        in_specs=[
            pl.BlockSpec((_BB, A, 1, h0, 128), lambda i: (i, 0, 0, 0, 0)),
            pl.BlockSpec((A, h1, 1, h1, 128), lambda i: (0, 0, 0, 0, 0)),
            pl.BlockSpec((A, h2, 1, h2, 128), lambda i: (0, 0, 0, 0, 0)),
        ],
        out_specs=pl.BlockSpec((1, 1), lambda i: (0, 0)),
    )(lg0, lg1, lg2)


def _final_kernel(acc_ref, xs_ref, vv_ref, out_ref):
    x = xs_ref[...]
    corr = vv_ref[...] * (POS_WEIGHT * _softplus(-x) - _softplus(x))
    total = acc_ref[...] + jnp.sum(corr, keepdims=True).reshape(1, 1)
    out_ref[...] = total / jnp.float32(N_TOTAL)


def _final(acc, xsel, val):
    return pl.pallas_call(
        _final_kernel,
        out_shape=jax.ShapeDtypeStruct((1, 1), jnp.float32),
        in_specs=[
            pl.BlockSpec((1, 1), lambda: (0, 0)),
            pl.BlockSpec((S * A * B, T), lambda: (0, 0)),
            pl.BlockSpec((S * A * B, T), lambda: (0, 0)),
        ],
        out_specs=pl.BlockSpec((1, 1), lambda: (0, 0)),
    )(acc, xsel, val)


def kernel(det0, det1, det2, scaled_anchors, targets):
    anch = scaled_anchors.reshape(1, S * A * 2)
    lg0, lg1, lg2, idx, val = _tables(det0.transpose(0, 1, 4, 2, 3),
                                      jnp.transpose(det1, (1, 2, 4, 3, 0)),
                                      jnp.transpose(det2, (1, 2, 4, 3, 0)),
                                      jnp.transpose(targets, (0, 2, 1)), anch)
    xsel = _gather(lg0.reshape(-1), lg1.reshape(-1), lg2.reshape(-1),
                   idx.reshape(S * A * B, T))
    acc = _sums(lg0, lg1, lg2)
    loss = _final(acc, xsel, val.reshape(S * A * B, T))
    return loss[0, 0]


# speedup vs baseline: 1.2783x; 1.0345x over previous
import jax
import jax.numpy as jnp
from jax import lax
from jax.experimental import pallas as pl
from jax.experimental.pallas import tpu as pltpu
from jax.experimental.pallas import tpu_sc as plsc

IOU_THRESHOLD = 0.5
POS_WEIGHT = 0.5

B, T, S, A = 64, 100, 3, 3
HS = (64, 32, 16)
N_TOTAL = B * A * sum(h * h for h in HS)
N_UPD = B * T * S * A
ROW = 120
N_ROWS = N_UPD // ROW
ROWS_PER_SCALE = B * T * A // ROW
NC, NS = 2, 16
ROWS_PER_WORKER = ROWS_PER_SCALE // (NC * NS)


def _softplus(x):
    return jnp.maximum(x, 0.0) + jnp.log1p(jnp.exp(-jnp.abs(x)))


def _prep_body(tgt_ref, anch_ref, idx_ref, val_ref):
    tgt = tgt_ref[...]
    tx, ty, tw, th = tgt[:, 0, :], tgt[:, 1, :], tgt[:, 2, :], tgt[:, 3, :]
    mask = (tx != -1.0) & (ty != -1.0) & (tw != -1.0) & (th != -1.0)
    biota = lax.broadcasted_iota(jnp.int32, (B, T), 0)

    zx, zy, zw, zh, ta, cy, cx, valid = [], [], [], [], [], [], [], []
    for s in range(S):
        h = float(HS[s])
        sx = tx * h
        sy = ty * h
        zx.append(sx - (jnp.floor(sx) + 0.5))
        zy.append(sy - (jnp.floor(sy) + 0.5))
        zw.append(tw * h)
        zh.append(th * h)
        ta.append((zw[s] - zx[s]) * (zh[s] - zy[s]))
        _cy = sy.astype(jnp.int32)
        _cx = sx.astype(jnp.int32)
        hi = HS[s]
        cy.append(_cy)
        cx.append(_cx)
        valid.append(mask & (_cy >= 0) & (_cy < hi) & (_cx >= 0) & (_cx < hi))

    dy = cy[0][:, :, None] ^ cy[0][:, None, :]
    dx = cx[0][:, :, None] ^ cx[0][:, None, :]
    later = (lax.broadcasted_iota(jnp.int32, (B, T, T), 2)
             > lax.broadcasted_iota(jnp.int32, (B, T, T), 1))
    live_base = []
    for s in range(S):
        same = ((dy >> s) | (dx >> s)) == 0
        killer = jnp.broadcast_to(valid[s][:, None, :], (B, T, T))
        killed = jnp.any(same & later & killer, axis=2)
        live_base.append(valid[s] & jnp.logical_not(killed))

    ious = []
    for s in range(S):
        for a in range(A):
            aw = anch_ref[0, (s * A + a) * 2]
            ah = anch_ref[0, (s * A + a) * 2 + 1]
            lx = jnp.maximum(-aw / 2, zx[s] - zw[s] / 2)
            ux = jnp.minimum(aw / 2, zx[s] + zw[s] / 2)
            ly = jnp.maximum(-ah / 2, zy[s] - zh[s] / 2)
            uy = jnp.minimum(ah / 2, zy[s] + zh[s] / 2)
            m = ((lx < ux).astype(jnp.float32) * (ly < uy).astype(jnp.float32))
            inter = (ux - lx) * (uy - ly) * m
            ious.append(inter / ((aw * ah + ta[s]) - inter))
    mx = ious[0]
    for z in ious[1:]:
        mx = jnp.maximum(mx, z)

    for s in range(S):
        hi = HS[s]
        for a in range(A):
            v = ious[s * A + a]
            v = jnp.where(v == mx, 1.0, v)
            v = jnp.where(v < IOU_THRESHOLD, 0.0, v)
            v = jnp.where((v != 0.0) & (v != 1.0), -1.0, v)
            if s == 0:
                addr = (((biota * A + a) * hi + cy[s]) * 128) + cx[s]
            else:
                addr = (((a * hi + cy[s]) * hi + cx[s]) * 128) + biota
            idx_ref[s * A + a] = jnp.where(valid[s], addr, 0)
            val_ref[s * A + a] = jnp.where(live_base[s], v, 0.0)


_CHUNK = 16
_NCHUNKS = (S * A * B) // _CHUNK


def _gather_body(d0_hbm, d1_hbm, d2_hbm, idx_hbm, xsel_hbm, idx_v, out_v, sem):
    c = lax.axis_index("c")
    sub = lax.axis_index("s")
    wid = sub * NC + c
    tables = (d0_hbm, d1_hbm, d2_hbm)
    for ck in range(_NCHUNKS):
        scale = ck // (_NCHUNKS // S)
        r0 = ck * _CHUNK

        @pl.when((wid == ck) | (wid == ck - (NC * NS)))
        def _(scale=scale, r0=r0):
            tbl = tables[scale]
            pltpu.sync_copy(idx_hbm.at[pl.ds(r0, _CHUNK)], idx_v)
            cps = [pltpu.async_copy(tbl.at[idx_v.at[r]], out_v.at[r], sem)
                   for r in range(_CHUNK)]
            for cp in cps:
                cp.wait()
            pltpu.sync_copy(out_v, xsel_hbm.at[pl.ds(r0, _CHUNK)])


def _gather(d0f, d1f, d2f, idx2):
    mesh = plsc.VectorSubcoreMesh(core_axis_name="c", subcore_axis_name="s")
    return pl.kernel(
        _gather_body,
        out_type=jax.ShapeDtypeStruct((S * A * B, T), jnp.float32),
        mesh=mesh,
        scratch_types=[
            pltpu.VMEM((_CHUNK, T), jnp.int32),
            pltpu.VMEM((_CHUNK, T), jnp.float32),
            pltpu.SemaphoreType.DMA,
        ],
    )(d0f, d1f, d2f, idx2)


_GRID = 2
_BB = B // _GRID


def _tables_kernel(d0_ref, d1_ref, d2_ref, tgt_ref, anch_ref, lg0_ref,
                   lg1_ref, lg2_ref, idx_ref, val_ref):
    i = pl.program_id(0)
    x = d0_ref[...]
    lg0_ref[:, :, :, :, 0:x.shape[4]] = x

    @pl.when(i == 0)
    def _():
        for ref, lg in ((d1_ref, lg1_ref), (d2_ref, lg2_ref)):
            y = ref[...]
            lg[:, :, :, :, 0:y.shape[4]] = y
        _prep_body(tgt_ref, anch_ref, idx_ref, val_ref)


def _tables(d0t, d1t, d2t, tgt, anch):
    h0, h1, h2 = HS
    return pl.pallas_call(
        _tables_kernel,
        grid=(_GRID,),
        out_shape=(
            jax.ShapeDtypeStruct((B, A, 1, h0, 128), jnp.float32),
            jax.ShapeDtypeStruct((A, h1, 1, h1, 128), jnp.float32),
            jax.ShapeDtypeStruct((A, h2, 1, h2, 128), jnp.float32),
            jax.ShapeDtypeStruct((S * A, B, T), jnp.int32),
            jax.ShapeDtypeStruct((S * A, B, T), jnp.float32),
        ),
        in_specs=[
            pl.BlockSpec((_BB, A, 1, h0, h0), lambda i: (i, 0, 4, 0, 0)),
            pl.BlockSpec((A, h1, 1, h1, B), lambda i: (0, 0, 4, 0, 0)),
            pl.BlockSpec((A, h2, 1, h2, B), lambda i: (0, 0, 4, 0, 0)),
            pl.BlockSpec((B, 4, T), lambda i: (0, 0, 0)),
            pl.BlockSpec(memory_space=pltpu.SMEM),
        ],
        out_specs=(
            pl.BlockSpec((_BB, A, 1, h0, 128), lambda i: (i, 0, 0, 0, 0)),
            pl.BlockSpec((A, h1, 1, h1, 128), lambda i: (0, 0, 0, 0, 0)),
            pl.BlockSpec((A, h2, 1, h2, 128), lambda i: (0, 0, 0, 0, 0)),
            pl.BlockSpec((S * A, B, T), lambda i: (0, 0, 0)),
            pl.BlockSpec((S * A, B, T), lambda i: (0, 0, 0)),
        ),
    )(d0t, d1t, d2t, tgt, anch)


def _sums_kernel(lg0_ref, lg1_ref, lg2_ref, acc_ref):
    i = pl.program_id(0)
    x = lg0_ref[...]
    lane = lax.broadcasted_iota(jnp.int32, x.shape, 4)
    acc = jnp.sum(jnp.where(lane < B, _softplus(x), 0.0),
                  keepdims=True).reshape(1, 1)

    @pl.when(i == 0)
    def _():
        extra = jnp.zeros((1, 1), jnp.float32)
        for ref in (lg1_ref, lg2_ref):
            y = ref[...]
            ln = lax.broadcasted_iota(jnp.int32, y.shape, 4)
            extra += jnp.sum(jnp.where(ln < B, _softplus(y), 0.0),
                             keepdims=True).reshape(1, 1)
        acc_ref[...] = extra

    acc_ref[...] += acc


def _sums(lg0, lg1, lg2):
    h0, h1, h2 = HS
    return pl.pallas_call(
        _sums_kernel,
        grid=(_GRID,),
        out_shape=jax.ShapeDtypeStruct((1, 1), jnp.float32),
        in_specs=[
            pl.BlockSpec((_BB, A, 1, h0, 128), lambda i: (i, 0, 0, 0, 0)),
            pl.BlockSpec((A, h1, 1, h1, 128), lambda i: (0, 0, 0, 0, 0)),
            pl.BlockSpec((A, h2, 1, h2, 128), lambda i: (0, 0, 0, 0, 0)),
        ],
        out_specs=pl.BlockSpec((1, 1), lambda i: (0, 0)),
    )(lg0, lg1, lg2)


def _final_kernel(acc_ref, xs_ref, vv_ref, out_ref):
    x = xs_ref[...]
    corr = vv_ref[...] * (POS_WEIGHT * _softplus(-x) - _softplus(x))
    total = acc_ref[...] + jnp.sum(corr, keepdims=True).reshape(1, 1)
    out_ref[...] = total / jnp.float32(N_TOTAL)


def _final(acc, xsel, val):
    return pl.pallas_call(
        _final_kernel,
        out_shape=jax.ShapeDtypeStruct((1, 1), jnp.float32),
        in_specs=[
            pl.BlockSpec((1, 1), lambda: (0, 0)),
            pl.BlockSpec((S * A * B, T), lambda: (0, 0)),
            pl.BlockSpec((S * A * B, T), lambda: (0, 0)),
        ],
        out_specs=pl.BlockSpec((1, 1), lambda: (0, 0)),
    )(acc, xsel, val)


def kernel(det0, det1, det2, scaled_anchors, targets):
    anch = scaled_anchors.reshape(1, S * A * 2)
    lg0, lg1, lg2, idx, val = _tables(det0.transpose(0, 1, 4, 2, 3),
                                      jnp.transpose(det1, (1, 2, 4, 3, 0)),
                                      jnp.transpose(det2, (1, 2, 4, 3, 0)),
                                      jnp.transpose(targets, (0, 2, 1)), anch)
    xsel = _gather(lg0.reshape(-1), lg1.reshape(-1), lg2.reshape(-1),
                   idx.reshape(S * A * B, T))
    acc = _sums(lg0, lg1, lg2)
    loss = _final(acc, xsel, val.reshape(S * A * B, T))
    return loss[0, 0]
